# baseline (speedup 1.0000x reference)
import functools

import jax
import jax.numpy as jnp
from jax import lax
from jax.experimental import pallas as pl
from jax.experimental.pallas import tpu as pltpu
from jax.experimental.pallas import tpu_sc as plsc

B = 16384
D = 64
NF = 5
CONCAT = NF * D
OUT = 128

_NC = 2
_NS = 16
_NW = _NC * _NS
_BPW = B // _NW
_CHUNK = 128
_NCHUNK = _BPW // _CHUNK


def _sc_gather_body(samp_hbm, m0, m1, m2, m3, m4, t0, t1, t2, t3, t4,
                    o0, o1, o2, o3, o4, samp_v, idx_v, rows_v, sem):
    wid = lax.axis_index("s") * _NC + lax.axis_index("c")
    base = wid * _BPW
    pltpu.sync_copy(samp_hbm.at[pl.ds(wid * _NCHUNK, _NCHUNK)], samp_v)
    maps = (m0, m1, m2, m3, m4)
    tabs = (t0, t1, t2, t3, t4)
    outs = (o0, o1, o2, o3, o4)
    for c in range(_NCHUNK):
        for f in range(NF):
            pltpu.async_copy(maps[f].at[samp_v.at[c]], idx_v, sem).wait()
            pltpu.async_copy(tabs[f].at[idx_v], rows_v, sem).wait()
            pltpu.sync_copy(rows_v, outs[f].at[pl.ds(base + c * _CHUNK, _CHUNK)])


_SC_MESH = plsc.VectorSubcoreMesh(core_axis_name="c", subcore_axis_name="s")

_sc_gather = functools.partial(
    pl.kernel,
    out_type=[jax.ShapeDtypeStruct((B, D), jnp.float32)] * NF,
    mesh=_SC_MESH,
    scratch_types=[
        pltpu.VMEM((_NCHUNK, _CHUNK), jnp.int32),
        pltpu.VMEM((_CHUNK,), jnp.int32),
        pltpu.VMEM((_CHUNK, D), jnp.float32),
        pltpu.SemaphoreType.DMA,
    ],
    compiler_params=pltpu.CompilerParams(use_tc_tiling_on_sc=False),
)(_sc_gather_body)


def _mlp_body(f0, f1, f2, f3, f4, w_ref, b_ref, o_ref):
    x = jnp.concatenate(
        [f0[...], f1[...], f2[...], f3[...], f4[...]], axis=-1)
    h = jnp.dot(x, w_ref[...], preferred_element_type=jnp.float32) + b_ref[...]
    o_ref[...] = h * (1.0 / (1.0 + jnp.exp(-h)))


def _mlp(feats, W, b2d):
    bm = 2048
    grid = (B // bm,)
    in_specs = [pl.BlockSpec((bm, D), lambda i: (i, 0)) for _ in range(NF)]
    in_specs += [
        pl.BlockSpec((CONCAT, OUT), lambda i: (0, 0)),
        pl.BlockSpec((1, OUT), lambda i: (0, 0)),
    ]
    return pl.pallas_call(
        _mlp_body,
        grid=grid,
        in_specs=in_specs,
        out_specs=pl.BlockSpec((bm, OUT), lambda i: (i, 0)),
        out_shape=jax.ShapeDtypeStruct((B, OUT), jnp.float32),
    )(*feats, W, b2d)


def kernel(sample, map_cat0, map_cat1, map_cat2, map_cat3,
           emb_cat0, emb_cat1, emb_cat2, emb_cat3,
           map_text, text_table, W, b):
    samp2d = sample.astype(jnp.int32).reshape(_NW * _NCHUNK, _CHUNK)
    feats = _sc_gather(
        samp2d,
        map_cat0.astype(jnp.int32), map_cat1.astype(jnp.int32),
        map_cat2.astype(jnp.int32), map_cat3.astype(jnp.int32),
        map_text.astype(jnp.int32),
        emb_cat0, emb_cat1, emb_cat2, emb_cat3, text_table,
    )
    return _mlp(feats, W, b.reshape(1, OUT))


# speedup vs baseline: 1.0775x; 1.0775x over previous
import functools

import jax
import jax.numpy as jnp
from jax import lax
from jax.experimental import pallas as pl
from jax.experimental.pallas import tpu as pltpu
from jax.experimental.pallas import tpu_sc as plsc

B = 16384
D = 64
NF = 5
XCOLS = 384
OUT = 128

_NC = 2
_NS = 16
_NW = _NC * _NS
_BPW = B // _NW
_CHUNK = 128
_NCHUNK = _BPW // _CHUNK
_NIT = _NCHUNK * NF
_NBUF = 8


def _sc_gather_body(samp_hbm, m0, m1, m2, m3, m4, t0, t1, t2, t3, t4,
                    oa_hbm, ob_hbm, oc_hbm, samp_v, idx_v, rows_v,
                    sem_m, sem_g, sem_w):
    wid = lax.axis_index("s") * _NC + lax.axis_index("c")
    base = wid * _BPW
    maps = (m0, m1, m2, m3, m4)
    tabs = (t0, t1, t2, t3, t4)
    pltpu.sync_copy(samp_hbm.at[pl.ds(wid * _NCHUNK, _NCHUNK)], samp_v)
    mdesc = []
    for i in range(_NIT):
        c, f = divmod(i, NF)
        mdesc.append(
            pltpu.async_copy(maps[f].at[samp_v.at[c]], idx_v.at[i], sem_m))

    def _write(j):
        c, f = divmod(j, NF)
        out = (oa_hbm, oa_hbm, ob_hbm, ob_hbm, oc_hbm)[f]
        col = (0, D, 0, D, 0)[f]
        rsl = pl.ds(base + c * _CHUNK, _CHUNK)
        w = [pltpu.async_copy(rows_v.at[j % _NBUF], out.at[rsl, pl.ds(col, D)],
                              sem_w)]
        if f == NF - 1:
            w.append(pltpu.async_copy(rows_v.at[j % _NBUF],
                                      oc_hbm.at[rsl, pl.ds(D, D)], sem_w))
        return w

    gdesc = [None] * _NIT
    wdesc = [None] * _NIT
    for i in range(_NIT):
        c, f = divmod(i, NF)
        if i >= _NBUF:
            for wd in wdesc[i - _NBUF]:
                wd.wait()
        mdesc[i].wait()
        gdesc[i] = pltpu.async_copy(tabs[f].at[idx_v.at[i]],
                                    rows_v.at[i % _NBUF], sem_g)
        if i >= 1:
            gdesc[i - 1].wait()
            wdesc[i - 1] = _write(i - 1)
    gdesc[_NIT - 1].wait()
    wdesc[_NIT - 1] = _write(_NIT - 1)
    for j in range(_NIT - _NBUF, _NIT):
        for wd in wdesc[j]:
            wd.wait()


_SC_MESH = plsc.VectorSubcoreMesh(core_axis_name="c", subcore_axis_name="s")

_sc_gather = functools.partial(
    pl.kernel,
    out_type=[jax.ShapeDtypeStruct((B, 2 * D), jnp.float32)] * 3,
    mesh=_SC_MESH,
    scratch_types=[
        pltpu.VMEM((_NCHUNK, _CHUNK), jnp.int32),
        pltpu.VMEM((_NIT, _CHUNK), jnp.int32),
        pltpu.VMEM((_NBUF, _CHUNK, D), jnp.float32),
        pltpu.SemaphoreType.DMA,
        pltpu.SemaphoreType.DMA,
        pltpu.SemaphoreType.DMA,
    ],
    compiler_params=pltpu.CompilerParams(use_tc_tiling_on_sc=False),
)(_sc_gather_body)


def _mlp_body(xa_ref, xb_ref, xc_ref, w_ref, b_ref, o_ref):
    x = jnp.concatenate([xa_ref[...], xb_ref[...], xc_ref[...]], axis=-1)
    h = jnp.dot(x, w_ref[...],
                preferred_element_type=jnp.float32) + b_ref[...]
    o_ref[...] = h * (1.0 / (1.0 + jnp.exp(-h)))


def _mlp(xa, xb, xc, w_pad, b2d):
    bm = 2048
    return pl.pallas_call(
        _mlp_body,
        grid=(B // bm,),
        in_specs=[
            pl.BlockSpec((bm, 2 * D), lambda i: (i, 0)),
            pl.BlockSpec((bm, 2 * D), lambda i: (i, 0)),
            pl.BlockSpec((bm, 2 * D), lambda i: (i, 0)),
            pl.BlockSpec((XCOLS, OUT), lambda i: (0, 0)),
            pl.BlockSpec((1, OUT), lambda i: (0, 0)),
        ],
        out_specs=pl.BlockSpec((bm, OUT), lambda i: (i, 0)),
        out_shape=jax.ShapeDtypeStruct((B, OUT), jnp.float32),
    )(xa, xb, xc, w_pad, b2d)


def kernel(sample, map_cat0, map_cat1, map_cat2, map_cat3,
           emb_cat0, emb_cat1, emb_cat2, emb_cat3,
           map_text, text_table, W, b):
    samp2d = sample.astype(jnp.int32).reshape(_NW * _NCHUNK, _CHUNK)
    xa, xb, xc = _sc_gather(
        samp2d,
        map_cat0.astype(jnp.int32), map_cat1.astype(jnp.int32),
        map_cat2.astype(jnp.int32), map_cat3.astype(jnp.int32),
        map_text.astype(jnp.int32),
        emb_cat0, emb_cat1, emb_cat2, emb_cat3, text_table,
    )
    w_pad = jnp.concatenate([W, jnp.zeros((D, OUT), dtype=W.dtype)], axis=0)
    return _mlp(xa, xb, xc, w_pad, b.reshape(1, OUT))
